# baseline (speedup 1.0000x reference)
import functools

import jax
import jax.numpy as jnp
from jax import lax
from jax.experimental import pallas as pl
from jax.experimental.pallas import tpu as pltpu
from jax.experimental.pallas import tpu_sc as plsc

B = 16384
F = 32
NW = 32
BPW = B // NW
NCHUNK = BPW // 128
NGROUP = BPW // 16


def _gmf_body(user_hbm, item_hbm, eu_hbm, ei_hbm, w_hbm, bias_hbm, out_hbm,
              uidx, iidx, urows, irows, wv, bv, outv, sem_u, sem_i):
    nc = 2
    wid = lax.axis_index("s") * nc + lax.axis_index("c")

    pltpu.sync_copy(user_hbm.at[pl.ds(wid * NCHUNK, NCHUNK)], uidx)
    pltpu.sync_copy(item_hbm.at[pl.ds(wid * NCHUNK, NCHUNK)], iidx)
    pltpu.sync_copy(w_hbm, wv)
    pltpu.sync_copy(bias_hbm, bv)

    copies = []
    for j in range(NCHUNK):
        copies.append(pltpu.async_copy(
            eu_hbm.at[uidx.at[j]], urows.at[pl.ds(j * 128, 128)], sem_u))
        copies.append(pltpu.async_copy(
            ei_hbm.at[iidx.at[j]], irows.at[pl.ds(j * 128, 128)], sem_i))
    for c in copies:
        c.wait()

    w_lo = wv[pl.ds(0, 16)]
    w_hi = wv[pl.ds(16, 16)]
    wcols = [w_lo[f] if f < 16 else w_hi[f - 16] for f in range(F)]
    bias = bv[...]

    def group(g, carry):
        row_idx = g * 16 + lax.iota(jnp.int32, 16)
        acc = bias
        for f in range(F):
            col_idx = jnp.full((16,), f, jnp.int32)
            u = plsc.load_gather(urows, [row_idx, col_idx])
            iv = plsc.load_gather(irows, [row_idx, col_idx])
            acc = acc + u * iv * wcols[f]
        outv[pl.ds(g * 16, 16)] = acc
        return carry

    lax.fori_loop(0, NGROUP, group, 0)

    pltpu.sync_copy(outv, out_hbm.at[pl.ds(wid * BPW, BPW)])


@functools.partial(jax.jit, static_argnames=())
def _gmf(user2, item2, embed_user, embed_item, w_vec, bias_vec):
    mesh = plsc.VectorSubcoreMesh(core_axis_name="c", subcore_axis_name="s",
                                  num_cores=2, num_subcores=16)
    kern = pl.kernel(
        _gmf_body,
        out_type=jax.ShapeDtypeStruct((B,), jnp.float32),
        mesh=mesh,
        compiler_params=pltpu.CompilerParams(needs_layout_passes=False,
                                             use_tc_tiling_on_sc=False),
        scratch_types=[
            pltpu.VMEM((NCHUNK, 128), jnp.int32),
            pltpu.VMEM((NCHUNK, 128), jnp.int32),
            pltpu.VMEM((BPW, F), jnp.float32),
            pltpu.VMEM((BPW, F), jnp.float32),
            pltpu.VMEM((F,), jnp.float32),
            pltpu.VMEM((16,), jnp.float32),
            pltpu.VMEM((BPW,), jnp.float32),
            pltpu.SemaphoreType.DMA,
            pltpu.SemaphoreType.DMA,
        ],
    )
    return kern(user2, item2, embed_user, embed_item, w_vec, bias_vec)


def kernel(user, item, embed_user, embed_item, W, b):
    user2 = user.reshape(NW * NCHUNK, 128)
    item2 = item.reshape(NW * NCHUNK, 128)
    w_vec = W.reshape(F)
    bias_vec = jnp.broadcast_to(b, (16,))
    return _gmf(user2, item2, embed_user, embed_item, w_vec, bias_vec)
